# baseline (speedup 1.0000x reference)
import jax
import jax.numpy as jnp
from jax import lax
from jax.experimental import pallas as pl
from jax.experimental.pallas import tpu as pltpu
from jax.experimental.pallas import tpu_sc as plsc

B = 16384
D = 64
NC = 2
NS = 16
NW = NC * NS
BPW = B // NW
CHUNK = 128
NCHUNK = BPW // CHUNK
L = 16


def _body(uidx_hbm, iidx_hbm, uemb_hbm, iemb_hbm, ubias_hbm, ibias_hbm,
          out_hbm, uidx_v, iidx_v, ue_v, ie_v, ub_v, ib_v, out_v, sems):
    wid = lax.axis_index("s") * NC + lax.axis_index("c")
    row0 = wid * NCHUNK

    pltpu.sync_copy(uidx_hbm.at[pl.ds(row0, NCHUNK)], uidx_v)
    pltpu.sync_copy(iidx_hbm.at[pl.ds(row0, NCHUNK)], iidx_v)

    descs = []
    for c in range(NCHUNK):
        s = pl.ds(c * CHUNK, CHUNK)
        descs.append((
            pltpu.async_copy(uemb_hbm.at[uidx_v.at[c]], ue_v.at[s], sems.at[c]),
            pltpu.async_copy(iemb_hbm.at[iidx_v.at[c]], ie_v.at[s], sems.at[c]),
            pltpu.async_copy(ubias_hbm.at[uidx_v.at[c]], ub_v.at[s], sems.at[c]),
            pltpu.async_copy(ibias_hbm.at[iidx_v.at[c]], ib_v.at[s], sems.at[c]),
        ))

    lanes = lax.iota(jnp.int32, L)
    for c in range(NCHUNK):
        for d_ in descs[c]:
            d_.wait()

        def grp_body(j, _, c=c):
            base = c * CHUNK + j * L
            vec = jnp.zeros((L,), jnp.float32)
            for t in range(L):
                r = base + t
                acc = ue_v[r, pl.ds(0, L)] * ie_v[r, pl.ds(0, L)]
                for k in range(1, D // L):
                    acc = acc + ue_v[r, pl.ds(k * L, L)] * ie_v[r, pl.ds(k * L, L)]
                vec = jnp.where(lanes == t, jnp.sum(acc), vec)
            out_v[pl.ds(base, L)] = vec
            return 0

        lax.fori_loop(0, CHUNK // L, grp_body, 0)

    def ep_body(j, _):
        s = pl.ds(j * L, L)
        r = out_v[s] + ub_v[s] + ib_v[s]
        out_v[s] = 5.0 / (1.0 + jnp.exp(-r))
        return 0

    lax.fori_loop(0, BPW // L, ep_body, 0)

    pltpu.sync_copy(out_v, out_hbm.at[pl.ds(wid * BPW, BPW)])


def kernel(x_batch, user_emb, item_emb, user_bias, item_bias):
    uidx = x_batch[:, 0].reshape(B // CHUNK, CHUNK)
    iidx = x_batch[:, 1].reshape(B // CHUNK, CHUNK)
    ub = user_bias.reshape(-1)
    ib = item_bias.reshape(-1)
    mesh = plsc.VectorSubcoreMesh(core_axis_name="c", subcore_axis_name="s")
    run = pl.kernel(
        _body,
        out_type=jax.ShapeDtypeStruct((B,), jnp.float32),
        mesh=mesh,
        compiler_params=pltpu.CompilerParams(
            needs_layout_passes=False, use_tc_tiling_on_sc=False
        ),
        scratch_types=[
            pltpu.VMEM((NCHUNK, CHUNK), jnp.int32),
            pltpu.VMEM((NCHUNK, CHUNK), jnp.int32),
            pltpu.VMEM((BPW, D), jnp.float32),
            pltpu.VMEM((BPW, D), jnp.float32),
            pltpu.VMEM((BPW,), jnp.float32),
            pltpu.VMEM((BPW,), jnp.float32),
            pltpu.VMEM((BPW,), jnp.float32),
            pltpu.SemaphoreType.DMA((NCHUNK,)),
        ],
    )
    return run(uidx, iidx, user_emb, item_emb, ub, ib)
